# Initial kernel scaffold; baseline (speedup 1.0000x reference)
import jax
import jax.numpy as jnp
from jax.experimental import pallas as pl


def kernel(features, edge_index, W1, b1, W2, b2):
    raise NotImplementedError("write your pallas kernel here")


# speedup vs baseline: 4.1805x; 4.1805x over previous
import functools

import jax
import jax.numpy as jnp
from jax import lax
from jax.experimental import pallas as pl
from jax.experimental.pallas import tpu as pltpu
from jax.experimental.pallas import tpu_sc as plsc

N = 10000
E = 320000
IN = 128
HID = 256
OUT = 128

NTILE = 16
EPT = E // NTILE
CHUNK = 80
NCH = EPT // CHUNK
NPAD = 10240
ROWS_PT = NPAD // NTILE

_mesh = plsc.VectorSubcoreMesh(core_axis_name="c", subcore_axis_name="s")


@functools.partial(
    pl.kernel,
    out_type=jax.ShapeDtypeStruct((2 * NPAD,), jnp.float32),
    mesh=_mesh,
    scratch_types=[
        pltpu.VMEM((CHUNK,), jnp.int32),
        pltpu.VMEM((CHUNK,), jnp.float32),
        pltpu.VMEM_SHARED((NPAD,), jnp.float32),
    ],
)
def _deg(src_hbm, dst_hbm, z_hbm, out_hbm, idx_v, ones_v, hist_sh):
    c = lax.axis_index("c")
    s = lax.axis_index("s")
    for k in range(CHUNK // 16):
        ones_v[pl.ds(k * 16, 16)] = jnp.ones((16,), jnp.float32)
    stripe = s * (NPAD // NTILE)
    pltpu.sync_copy(z_hbm.at[pl.ds(stripe, NPAD // NTILE)],
                    hist_sh.at[pl.ds(stripe, NPAD // NTILE)])
    plsc.subcore_barrier()

    def count(edge_hbm):
        def body(g, carry):
            base = s * EPT + g * CHUNK
            pltpu.sync_copy(edge_hbm.at[pl.ds(base, CHUNK)], idx_v)
            pltpu.sync_copy(ones_v, hist_sh.at[idx_v], add=True)
            return carry
        lax.fori_loop(0, NCH, body, 0)

    @pl.when(c == 0)
    def _():
        count(src_hbm)

    @pl.when(c == 1)
    def _():
        count(dst_hbm)

    plsc.subcore_barrier()
    pltpu.sync_copy(hist_sh.at[pl.ds(stripe, NPAD // NTILE)],
                    out_hbm.at[pl.ds(c * NPAD + stripe, NPAD // NTILE)])


@functools.partial(
    pl.kernel,
    out_type=jax.ShapeDtypeStruct((2 * NPAD, HID // 2), jnp.float32),
    mesh=_mesh,
    scratch_types=[
        pltpu.VMEM((CHUNK,), jnp.int32),
        pltpu.VMEM((CHUNK,), jnp.int32),
        pltpu.VMEM((CHUNK, HID // 2), jnp.float32),
        pltpu.VMEM_SHARED((NPAD, HID // 2), jnp.float32),
        pltpu.SemaphoreType.DMA,
    ],
)
def _agg_l1(h_hbm, src_hbm, dst_hbm, z_hbm, out_hbm,
            idxs_v, idxd_v, rows_v, acc_sh, sem):
    c = lax.axis_index("c")
    s = lax.axis_index("s")
    r0 = s * ROWS_PT
    pltpu.sync_copy(z_hbm.at[pl.ds(r0, ROWS_PT)],
                    acc_sh.at[pl.ds(r0, ROWS_PT)])
    plsc.subcore_barrier()
    off = c * N

    def body(g, carry):
        base = s * EPT + g * CHUNK
        pltpu.sync_copy(src_hbm.at[pl.ds(base, CHUNK)], idxs_v)
        pltpu.sync_copy(dst_hbm.at[pl.ds(base, CHUNK)], idxd_v)
        for k in range(CHUNK // 16):
            sl = pl.ds(k * 16, 16)
            idxs_v[sl] = idxs_v[sl] + off
        pltpu.async_copy(h_hbm.at[idxs_v], rows_v, sem).wait()
        pltpu.sync_copy(rows_v, acc_sh.at[idxd_v], add=True)
        return carry

    lax.fori_loop(0, NCH, body, 0)
    plsc.subcore_barrier()
    pltpu.sync_copy(acc_sh.at[pl.ds(r0, ROWS_PT)],
                    out_hbm.at[pl.ds(c * NPAD + r0, ROWS_PT)])


EPT2 = E // (2 * NTILE)
NCH2 = EPT2 // CHUNK


@functools.partial(
    pl.kernel,
    out_type=jax.ShapeDtypeStruct((2 * NPAD, OUT), jnp.float32),
    mesh=_mesh,
    scratch_types=[
        pltpu.VMEM((CHUNK,), jnp.int32),
        pltpu.VMEM((CHUNK,), jnp.int32),
        pltpu.VMEM((CHUNK, OUT), jnp.float32),
        pltpu.VMEM_SHARED((NPAD, OUT), jnp.float32),
        pltpu.SemaphoreType.DMA,
    ],
)
def _agg_l2(h_hbm, src_hbm, dst_hbm, z_hbm, out_hbm,
            idxs_v, idxd_v, rows_v, acc_sh, sem):
    c = lax.axis_index("c")
    s = lax.axis_index("s")
    r0 = s * ROWS_PT
    pltpu.sync_copy(z_hbm.at[pl.ds(r0, ROWS_PT)],
                    acc_sh.at[pl.ds(r0, ROWS_PT)])
    plsc.subcore_barrier()

    def body(g, carry):
        base = (c * NTILE + s) * EPT2 + g * CHUNK
        pltpu.sync_copy(src_hbm.at[pl.ds(base, CHUNK)], idxs_v)
        pltpu.sync_copy(dst_hbm.at[pl.ds(base, CHUNK)], idxd_v)
        pltpu.async_copy(h_hbm.at[idxs_v], rows_v, sem).wait()
        pltpu.sync_copy(rows_v, acc_sh.at[idxd_v], add=True)
        return carry

    lax.fori_loop(0, NCH2, body, 0)
    plsc.subcore_barrier()
    pltpu.sync_copy(acc_sh.at[pl.ds(r0, ROWS_PT)],
                    out_hbm.at[pl.ds(c * NPAD + r0, ROWS_PT)])


_BLK = 1000


def _b1_body(x_ref, w_ref, deg_ref, out_ref):
    h = jnp.dot(x_ref[...], w_ref[...], preferred_element_type=jnp.float32)
    ns = lax.rsqrt(jnp.maximum(deg_ref[...], 1.0))
    out_ref[0] = h[:, :HID // 2] * ns
    out_ref[1] = h[:, HID // 2:] * ns


def _b1(x, w1, deg_out):
    return pl.pallas_call(
        _b1_body,
        grid=(N // _BLK,),
        in_specs=[
            pl.BlockSpec((_BLK, IN), lambda i: (i, 0)),
            pl.BlockSpec((IN, HID), lambda i: (0, 0)),
            pl.BlockSpec((_BLK, 1), lambda i: (i, 0)),
        ],
        out_specs=pl.BlockSpec((2, _BLK, HID // 2), lambda i: (0, i, 0)),
        out_shape=jax.ShapeDtypeStruct((2, N, HID // 2), jnp.float32),
    )(x, w1, deg_out)


def _b2_body(a_ref, di_ref, do_ref, b1_ref, w2a_ref, w2b_ref, out_ref):
    nd = lax.rsqrt(jnp.maximum(di_ref[...], 1.0))
    ns = lax.rsqrt(jnp.maximum(do_ref[...], 1.0))
    b = b1_ref[...]
    u0 = jnp.maximum(a_ref[0] * nd + b[:, :HID // 2], 0.0)
    u1 = jnp.maximum(a_ref[1] * nd + b[:, HID // 2:], 0.0)
    h2 = (jnp.dot(u0, w2a_ref[...], preferred_element_type=jnp.float32)
          + jnp.dot(u1, w2b_ref[...], preferred_element_type=jnp.float32))
    out_ref[...] = h2 * ns


def _b2(agg1, deg_in, deg_out, b1, w2a, w2b):
    return pl.pallas_call(
        _b2_body,
        grid=(N // _BLK,),
        in_specs=[
            pl.BlockSpec((2, _BLK, HID // 2), lambda i: (0, i, 0)),
            pl.BlockSpec((_BLK, 1), lambda i: (i, 0)),
            pl.BlockSpec((_BLK, 1), lambda i: (i, 0)),
            pl.BlockSpec((1, HID), lambda i: (0, 0)),
            pl.BlockSpec((HID // 2, OUT), lambda i: (0, 0)),
            pl.BlockSpec((HID // 2, OUT), lambda i: (0, 0)),
        ],
        out_specs=pl.BlockSpec((_BLK, OUT), lambda i: (i, 0)),
        out_shape=jax.ShapeDtypeStruct((N, OUT), jnp.float32),
    )(agg1, deg_in, deg_out, b1, w2a, w2b)


def _b3_body(a_ref, di_ref, b2_ref, out_ref):
    nd = lax.rsqrt(jnp.maximum(di_ref[...], 1.0))
    out_ref[...] = (a_ref[0] + a_ref[1]) * nd + b2_ref[...]


def _b3(agg2, deg_in, b2):
    return pl.pallas_call(
        _b3_body,
        grid=(N // _BLK,),
        in_specs=[
            pl.BlockSpec((2, _BLK, OUT), lambda i: (0, i, 0)),
            pl.BlockSpec((_BLK, 1), lambda i: (i, 0)),
            pl.BlockSpec((1, OUT), lambda i: (0, 0)),
        ],
        out_specs=pl.BlockSpec((_BLK, OUT), lambda i: (i, 0)),
        out_shape=jax.ShapeDtypeStruct((N, OUT), jnp.float32),
    )(agg2, deg_in, b2)


def kernel(features, edge_index, W1, b1, W2, b2):
    src = edge_index[0].astype(jnp.int32)
    dst = edge_index[1].astype(jnp.int32)

    degs = _deg(src, dst, jnp.zeros((NPAD,), jnp.float32))
    deg_out = degs[0:N].reshape(N, 1)
    deg_in = degs[NPAD:NPAD + N].reshape(N, 1)

    h1 = _b1(features, W1, deg_out).reshape(2 * N, HID // 2)
    agg1 = _agg_l1(h1, src, dst, jnp.zeros((NPAD, HID // 2), jnp.float32))

    h2 = _b2(agg1.reshape(2, NPAD, HID // 2), deg_in, deg_out,
             b1.reshape(1, HID), W2[:HID // 2], W2[HID // 2:])
    agg2 = _agg_l2(h2, src, dst, jnp.zeros((NPAD, OUT), jnp.float32))

    return _b3(agg2.reshape(2, NPAD, OUT), deg_in, b2.reshape(1, OUT))


# speedup vs baseline: 8.5286x; 2.0401x over previous
import functools

import jax
import jax.numpy as jnp
from jax import lax
from jax.experimental import pallas as pl
from jax.experimental.pallas import tpu as pltpu
from jax.experimental.pallas import tpu_sc as plsc

N = 10000
E = 320000
IN = 128
HID = 256
OUT = 128

NTILE = 16
EPT = E // NTILE
CHUNK = 80
NCH = EPT // CHUNK
NPAD = 10240
ROWS_PT = NPAD // NTILE

_mesh = plsc.VectorSubcoreMesh(core_axis_name="c", subcore_axis_name="s")


@functools.partial(
    pl.kernel,
    out_type=jax.ShapeDtypeStruct((2 * NPAD,), jnp.float32),
    mesh=_mesh,
    scratch_types=[
        pltpu.VMEM((CHUNK,), jnp.int32),
        pltpu.VMEM((CHUNK,), jnp.float32),
        pltpu.VMEM_SHARED((NPAD,), jnp.float32),
    ],
)
def _deg(src_hbm, dst_hbm, z_hbm, out_hbm, idx_v, ones_v, hist_sh):
    c = lax.axis_index("c")
    s = lax.axis_index("s")
    for k in range(CHUNK // 16):
        ones_v[pl.ds(k * 16, 16)] = jnp.ones((16,), jnp.float32)
    stripe = s * (NPAD // NTILE)
    pltpu.sync_copy(z_hbm.at[pl.ds(stripe, NPAD // NTILE)],
                    hist_sh.at[pl.ds(stripe, NPAD // NTILE)])
    plsc.subcore_barrier()

    def count(edge_hbm):
        def body(g, carry):
            base = s * EPT + g * CHUNK
            pltpu.sync_copy(edge_hbm.at[pl.ds(base, CHUNK)], idx_v)
            pltpu.sync_copy(ones_v, hist_sh.at[idx_v], add=True)
            return carry
        lax.fori_loop(0, NCH, body, 0)

    @pl.when(c == 0)
    def _():
        count(src_hbm)

    @pl.when(c == 1)
    def _():
        count(dst_hbm)

    plsc.subcore_barrier()
    pltpu.sync_copy(hist_sh.at[pl.ds(stripe, NPAD // NTILE)],
                    out_hbm.at[pl.ds(c * NPAD + stripe, NPAD // NTILE)])


BPB = 8
ITC = 2 * BPB


def _edge_pipeline(tab, acc_sh, src3, dst3, p, nch, nbt,
                   a_s, a_d, b_s, b_d, rows_v, sem_a, sem_b, gsems, ssems):
    def idx_block(bufs, bufd, row0, sem):
        return (pltpu.make_async_copy(src3.at[p, pl.ds(row0, BPB)],
                                      bufs, sem),
                pltpu.make_async_copy(dst3.at[p, pl.ds(row0, BPB)],
                                      bufd, sem))

    def start_block(bufs, bufd, row0, sem):
        x, y = idx_block(bufs, bufd, row0, sem)
        x.start()
        y.start()

    def wait_block(bufs, bufd, row0, sem):
        x, y = idx_block(bufs, bufd, row0, sem)
        x.wait()
        y.wait()

    def gath(s_slice, r):
        return pltpu.make_async_copy(tab.at[s_slice], rows_v.at[r],
                                     gsems[r])

    def scat(d_slice, r):
        return pltpu.make_async_copy(rows_v.at[r], acc_sh.at[d_slice],
                                     ssems[r])

    start_block(a_s, a_d, 0, sem_a)
    slices = ([(a_s.at[i], a_d.at[i]) for i in range(BPB)]
              + [(b_s.at[i], b_d.at[i]) for i in range(BPB)])

    def body(k, carry):
        g0 = k * ITC
        wait_block(a_s, a_d, g0, sem_a)
        for h in range(ITC):
            g = g0 + h
            r = h % 2
            if h == 2:
                start_block(b_s, b_d, g0 + BPB, sem_b)
            if h == BPB:
                wait_block(b_s, b_d, g0 + BPB, sem_b)
            if h == BPB + 2:
                @pl.when(k + 1 < nbt)
                def _():
                    start_block(a_s, a_d, g0 + ITC, sem_a)

            ps = slices[h][0]

            @pl.when(g < nch)
            def _(ps=ps, r=r):
                gath(ps, r).start()

            qs, qd = slices[h - 1] if h >= 1 else slices[ITC - 1]
            rq = (h - 1) % 2

            @pl.when(jnp.logical_and(g >= 1, g - 1 < nch))
            def _(qs=qs, qd=qd, rq=rq):
                gath(qs, rq).wait()
                d = scat(qd, rq)
                d.start(add=True)
                d.wait()

        return carry

    lax.fori_loop(0, nbt, body, 0)


NCHT = EPT // CHUNK
NBT1 = 16
NCHP1 = NBT1 * ITC


@functools.partial(
    pl.kernel,
    out_type=jax.ShapeDtypeStruct((2 * NPAD, HID // 2), jnp.float32),
    mesh=_mesh,
    scratch_types=[
        pltpu.VMEM((BPB, CHUNK), jnp.int32),
        pltpu.VMEM((BPB, CHUNK), jnp.int32),
        pltpu.VMEM((BPB, CHUNK), jnp.int32),
        pltpu.VMEM((BPB, CHUNK), jnp.int32),
        pltpu.VMEM((2, CHUNK, HID // 2), jnp.float32),
        pltpu.VMEM_SHARED((NPAD, HID // 2), jnp.float32),
        pltpu.SemaphoreType.DMA,
        pltpu.SemaphoreType.DMA,
        pltpu.SemaphoreType.DMA,
        pltpu.SemaphoreType.DMA,
        pltpu.SemaphoreType.DMA,
        pltpu.SemaphoreType.DMA,
    ],
)
def _agg_l1(h_hbm, src3_hbm, dst3_hbm, z_hbm, out_hbm,
            a_s, a_d, b_s, b_d, rows_v, acc_sh,
            sem_a, sem_b, gs0, gs1, ss0, ss1):
    c = lax.axis_index("c")
    s = lax.axis_index("s")
    r0 = s * ROWS_PT
    pltpu.sync_copy(z_hbm.at[pl.ds(r0, ROWS_PT)],
                    acc_sh.at[pl.ds(r0, ROWS_PT)])
    plsc.subcore_barrier()
    _edge_pipeline(h_hbm.at[c], acc_sh, src3_hbm, dst3_hbm, s, NCHT, NBT1,
                   a_s, a_d, b_s, b_d, rows_v, sem_a, sem_b,
                   (gs0, gs1), (ss0, ss1))
    plsc.subcore_barrier()
    pltpu.sync_copy(acc_sh.at[pl.ds(r0, ROWS_PT)],
                    out_hbm.at[pl.ds(c * NPAD + r0, ROWS_PT)])


NPART = 2 * NTILE
EPT2 = E // NPART
NCHT2 = EPT2 // CHUNK
NBT2 = 8
NCHP2 = NBT2 * ITC


@functools.partial(
    pl.kernel,
    out_type=jax.ShapeDtypeStruct((2 * NPAD, OUT), jnp.float32),
    mesh=_mesh,
    scratch_types=[
        pltpu.VMEM((BPB, CHUNK), jnp.int32),
        pltpu.VMEM((BPB, CHUNK), jnp.int32),
        pltpu.VMEM((BPB, CHUNK), jnp.int32),
        pltpu.VMEM((BPB, CHUNK), jnp.int32),
        pltpu.VMEM((2, CHUNK, OUT), jnp.float32),
        pltpu.VMEM_SHARED((NPAD, OUT), jnp.float32),
        pltpu.SemaphoreType.DMA,
        pltpu.SemaphoreType.DMA,
        pltpu.SemaphoreType.DMA,
        pltpu.SemaphoreType.DMA,
        pltpu.SemaphoreType.DMA,
        pltpu.SemaphoreType.DMA,
    ],
)
def _agg_l2(h_hbm, src3_hbm, dst3_hbm, z_hbm, out_hbm,
            a_s, a_d, b_s, b_d, rows_v, acc_sh,
            sem_a, sem_b, gs0, gs1, ss0, ss1):
    c = lax.axis_index("c")
    s = lax.axis_index("s")
    r0 = s * ROWS_PT
    pltpu.sync_copy(z_hbm.at[pl.ds(r0, ROWS_PT)],
                    acc_sh.at[pl.ds(r0, ROWS_PT)])
    plsc.subcore_barrier()
    _edge_pipeline(h_hbm, acc_sh, src3_hbm, dst3_hbm, c * NTILE + s,
                   NCHT2, NBT2, a_s, a_d, b_s, b_d, rows_v, sem_a, sem_b,
                   (gs0, gs1), (ss0, ss1))
    plsc.subcore_barrier()
    pltpu.sync_copy(acc_sh.at[pl.ds(r0, ROWS_PT)],
                    out_hbm.at[pl.ds(c * NPAD + r0, ROWS_PT)])


_BLK = 1000


def _b1_body(x_ref, w_ref, deg_ref, out_ref):
    h = jnp.dot(x_ref[...], w_ref[...], preferred_element_type=jnp.float32)
    ns = lax.rsqrt(jnp.maximum(deg_ref[...], 1.0))
    out_ref[0] = h[:, :HID // 2] * ns
    out_ref[1] = h[:, HID // 2:] * ns


def _b1(x, w1, deg_out):
    return pl.pallas_call(
        _b1_body,
        grid=(N // _BLK,),
        in_specs=[
            pl.BlockSpec((_BLK, IN), lambda i: (i, 0)),
            pl.BlockSpec((IN, HID), lambda i: (0, 0)),
            pl.BlockSpec((_BLK, 1), lambda i: (i, 0)),
        ],
        out_specs=pl.BlockSpec((2, _BLK, HID // 2), lambda i: (0, i, 0)),
        out_shape=jax.ShapeDtypeStruct((2, N, HID // 2), jnp.float32),
    )(x, w1, deg_out)


def _b2_body(a_ref, di_ref, do_ref, b1_ref, w2a_ref, w2b_ref, out_ref):
    nd = lax.rsqrt(jnp.maximum(di_ref[...], 1.0))
    ns = lax.rsqrt(jnp.maximum(do_ref[...], 1.0))
    b = b1_ref[...]
    u0 = jnp.maximum(a_ref[0] * nd + b[:, :HID // 2], 0.0)
    u1 = jnp.maximum(a_ref[1] * nd + b[:, HID // 2:], 0.0)
    h2 = (jnp.dot(u0, w2a_ref[...], preferred_element_type=jnp.float32)
          + jnp.dot(u1, w2b_ref[...], preferred_element_type=jnp.float32))
    out_ref[...] = h2 * ns


def _b2(agg1, deg_in, deg_out, b1, w2a, w2b):
    return pl.pallas_call(
        _b2_body,
        grid=(N // _BLK,),
        in_specs=[
            pl.BlockSpec((2, _BLK, HID // 2), lambda i: (0, i, 0)),
            pl.BlockSpec((_BLK, 1), lambda i: (i, 0)),
            pl.BlockSpec((_BLK, 1), lambda i: (i, 0)),
            pl.BlockSpec((1, HID), lambda i: (0, 0)),
            pl.BlockSpec((HID // 2, OUT), lambda i: (0, 0)),
            pl.BlockSpec((HID // 2, OUT), lambda i: (0, 0)),
        ],
        out_specs=pl.BlockSpec((_BLK, OUT), lambda i: (i, 0)),
        out_shape=jax.ShapeDtypeStruct((N, OUT), jnp.float32),
    )(agg1, deg_in, deg_out, b1, w2a, w2b)


def _b3_body(a_ref, di_ref, b2_ref, out_ref):
    nd = lax.rsqrt(jnp.maximum(di_ref[...], 1.0))
    out_ref[...] = (a_ref[0] + a_ref[1]) * nd + b2_ref[...]


def _b3(agg2, deg_in, b2):
    return pl.pallas_call(
        _b3_body,
        grid=(N // _BLK,),
        in_specs=[
            pl.BlockSpec((2, _BLK, OUT), lambda i: (0, i, 0)),
            pl.BlockSpec((_BLK, 1), lambda i: (i, 0)),
            pl.BlockSpec((1, OUT), lambda i: (0, 0)),
        ],
        out_specs=pl.BlockSpec((_BLK, OUT), lambda i: (i, 0)),
        out_shape=jax.ShapeDtypeStruct((N, OUT), jnp.float32),
    )(agg2, deg_in, b2)


def kernel(features, edge_index, W1, b1, W2, b2):
    src = edge_index[0].astype(jnp.int32)
    dst = edge_index[1].astype(jnp.int32)

    degs = _deg(src, dst, jnp.zeros((NPAD,), jnp.float32))
    deg_out = degs[0:N].reshape(N, 1)
    deg_in = degs[NPAD:NPAD + N].reshape(N, 1)

    src3_l1 = jnp.pad(src.reshape(NTILE, NCHT, CHUNK),
                      ((0, 0), (0, NCHP1 - NCHT), (0, 0)))
    dst3_l1 = jnp.pad(dst.reshape(NTILE, NCHT, CHUNK),
                      ((0, 0), (0, NCHP1 - NCHT), (0, 0)))
    src3_l2 = jnp.pad(src.reshape(NPART, NCHT2, CHUNK),
                      ((0, 0), (0, NCHP2 - NCHT2), (0, 0)))
    dst3_l2 = jnp.pad(dst.reshape(NPART, NCHT2, CHUNK),
                      ((0, 0), (0, NCHP2 - NCHT2), (0, 0)))

    h1 = _b1(features, W1, deg_out)
    agg1 = _agg_l1(h1, src3_l1, dst3_l1,
                   jnp.zeros((NPAD, HID // 2), jnp.float32))

    h2 = _b2(agg1.reshape(2, NPAD, HID // 2), deg_in, deg_out,
             b1.reshape(1, HID), W2[:HID // 2], W2[HID // 2:])
    agg2 = _agg_l2(h2, src3_l2, dst3_l2, jnp.zeros((NPAD, OUT), jnp.float32))

    return _b3(agg2.reshape(2, NPAD, OUT), deg_in, b2.reshape(1, OUT))


# speedup vs baseline: 10.3075x; 1.2086x over previous
import functools

import jax
import jax.numpy as jnp
from jax import lax
from jax.experimental import pallas as pl
from jax.experimental.pallas import tpu as pltpu
from jax.experimental.pallas import tpu_sc as plsc

N = 10000
E = 320000
IN = 128
HID = 256
OUT = 128

NTILE = 16
EPT = E // NTILE
CHUNK = 80
NCH = EPT // CHUNK
NPAD = 10240
ROWS_PT = NPAD // NTILE

_mesh = plsc.VectorSubcoreMesh(core_axis_name="c", subcore_axis_name="s")


NHIST = 4
DEG_BPB = 8
DEG_ITC = 2 * DEG_BPB
DEG_NBT = 16


@functools.partial(
    pl.kernel,
    out_type=jax.ShapeDtypeStruct((2 * NPAD,), jnp.float32),
    mesh=_mesh,
    scratch_types=[
        pltpu.VMEM((DEG_BPB, CHUNK), jnp.int32),
        pltpu.VMEM((DEG_BPB, CHUNK), jnp.int32),
        pltpu.VMEM((CHUNK,), jnp.float32),
        pltpu.VMEM((NHIST, ROWS_PT), jnp.float32),
        pltpu.VMEM((ROWS_PT,), jnp.float32),
        pltpu.VMEM_SHARED((NPAD,), jnp.float32),
        pltpu.VMEM_SHARED((NPAD,), jnp.float32),
        pltpu.VMEM_SHARED((NPAD,), jnp.float32),
        pltpu.VMEM_SHARED((NPAD,), jnp.float32),
        pltpu.SemaphoreType.DMA,
        pltpu.SemaphoreType.DMA,
        pltpu.SemaphoreType.DMA,
        pltpu.SemaphoreType.DMA,
        pltpu.SemaphoreType.DMA,
        pltpu.SemaphoreType.DMA,
    ],
)
def _deg(edges4_hbm, z_hbm, out_hbm, a_i, b_i, ones_v, part_v, red_v,
         h0, h1, h2, h3, sem_a, sem_b, ss0, ss1, ss2, ss3):
    c = lax.axis_index("c")
    s = lax.axis_index("s")
    hists = (h0, h1, h2, h3)
    ssems = (ss0, ss1, ss2, ss3)
    for k in range(CHUNK // 16):
        ones_v[pl.ds(k * 16, 16)] = jnp.ones((16,), jnp.float32)
    stripe = s * ROWS_PT
    for j in range(NHIST):
        pltpu.sync_copy(z_hbm.at[pl.ds(stripe, ROWS_PT)],
                        hists[j].at[pl.ds(stripe, ROWS_PT)])
    plsc.subcore_barrier()

    def idx_block(buf, row0, sem):
        return pltpu.make_async_copy(edges4_hbm.at[c, s].at[pl.ds(row0,
                                                                  DEG_BPB)],
                                     buf, sem)

    def scat(idx_slice, j):
        return pltpu.make_async_copy(ones_v, hists[j].at[idx_slice],
                                     ssems[j])

    idx_block(a_i, 0, sem_a).start()
    slices = ([a_i.at[i] for i in range(DEG_BPB)]
              + [b_i.at[i] for i in range(DEG_BPB)])

    def body(k, carry):
        g0 = k * DEG_ITC
        idx_block(a_i, g0, sem_a).wait()
        for h in range(DEG_ITC):
            g = g0 + h
            j = h % NHIST
            pd = slices[h - NHIST] if h >= NHIST else slices[DEG_ITC
                                                             - NHIST + h]

            @pl.when(jnp.logical_and(g >= NHIST, g - NHIST < NCH))
            def _(pd=pd, j=j):
                scat(pd, j).wait()

            if h == NHIST:
                idx_block(b_i, g0 + DEG_BPB, sem_b).start()
            if h == DEG_BPB:
                idx_block(b_i, g0 + DEG_BPB, sem_b).wait()
            if h == DEG_BPB + NHIST:
                @pl.when(k + 1 < DEG_NBT)
                def _():
                    idx_block(a_i, g0 + DEG_ITC, sem_a).start()

            ps = slices[h]

            @pl.when(g < NCH)
            def _(ps=ps, j=j):
                scat(ps, j).start(add=True)

        return carry

    lax.fori_loop(0, DEG_NBT, body, 0)
    plsc.subcore_barrier()
    for j in range(NHIST):
        pltpu.sync_copy(hists[j].at[pl.ds(stripe, ROWS_PT)],
                        part_v.at[j])

    def red(i, carry):
        sl = pl.ds(i * 16, 16)
        red_v[sl] = (part_v[0, sl] + part_v[1, sl]
                     + part_v[2, sl] + part_v[3, sl])
        return carry

    lax.fori_loop(0, ROWS_PT // 16, red, 0)
    pltpu.sync_copy(red_v, out_hbm.at[pl.ds(c * NPAD + stripe, ROWS_PT)])


BPB = 8
ITC = 2 * BPB


def _edge_pipeline(tab, acc_sh, src3, dst3, p, nch, nbt,
                   a_s, a_d, b_s, b_d, rows_v, sem_a, sem_b, gsems, ssems):
    def idx_block(bufs, bufd, row0, sem):
        return (pltpu.make_async_copy(src3.at[p, pl.ds(row0, BPB)],
                                      bufs, sem),
                pltpu.make_async_copy(dst3.at[p, pl.ds(row0, BPB)],
                                      bufd, sem))

    def start_block(bufs, bufd, row0, sem):
        x, y = idx_block(bufs, bufd, row0, sem)
        x.start()
        y.start()

    def wait_block(bufs, bufd, row0, sem):
        x, y = idx_block(bufs, bufd, row0, sem)
        x.wait()
        y.wait()

    def gath(s_slice, r):
        return pltpu.make_async_copy(tab.at[s_slice], rows_v.at[r],
                                     gsems[r])

    def scat(d_slice, r):
        return pltpu.make_async_copy(rows_v.at[r], acc_sh.at[d_slice],
                                     ssems[r])

    start_block(a_s, a_d, 0, sem_a)
    slices = ([(a_s.at[i], a_d.at[i]) for i in range(BPB)]
              + [(b_s.at[i], b_d.at[i]) for i in range(BPB)])

    def body(k, carry):
        g0 = k * ITC
        wait_block(a_s, a_d, g0, sem_a)
        for h in range(ITC):
            g = g0 + h
            r = h % 2
            if h == 2:
                start_block(b_s, b_d, g0 + BPB, sem_b)
            if h == BPB:
                wait_block(b_s, b_d, g0 + BPB, sem_b)
            if h == BPB + 2:
                @pl.when(k + 1 < nbt)
                def _():
                    start_block(a_s, a_d, g0 + ITC, sem_a)

            ps = slices[h][0]

            @pl.when(g < nch)
            def _(ps=ps, r=r):
                gath(ps, r).start()

            qs, qd = slices[h - 1] if h >= 1 else slices[ITC - 1]
            rq = (h - 1) % 2

            @pl.when(jnp.logical_and(g >= 1, g - 1 < nch))
            def _(qs=qs, qd=qd, rq=rq):
                gath(qs, rq).wait()
                d = scat(qd, rq)
                d.start(add=True)
                d.wait()

        return carry

    lax.fori_loop(0, nbt, body, 0)


NCHT = EPT // CHUNK
NBT1 = 16
NCHP1 = NBT1 * ITC


@functools.partial(
    pl.kernel,
    out_type=jax.ShapeDtypeStruct((2 * NPAD, HID // 2), jnp.float32),
    mesh=_mesh,
    scratch_types=[
        pltpu.VMEM((BPB, CHUNK), jnp.int32),
        pltpu.VMEM((BPB, CHUNK), jnp.int32),
        pltpu.VMEM((BPB, CHUNK), jnp.int32),
        pltpu.VMEM((BPB, CHUNK), jnp.int32),
        pltpu.VMEM((2, CHUNK, HID // 2), jnp.float32),
        pltpu.VMEM_SHARED((NPAD, HID // 2), jnp.float32),
        pltpu.SemaphoreType.DMA,
        pltpu.SemaphoreType.DMA,
        pltpu.SemaphoreType.DMA,
        pltpu.SemaphoreType.DMA,
        pltpu.SemaphoreType.DMA,
        pltpu.SemaphoreType.DMA,
    ],
)
def _agg_l1(h_hbm, src3_hbm, dst3_hbm, z_hbm, out_hbm,
            a_s, a_d, b_s, b_d, rows_v, acc_sh,
            sem_a, sem_b, gs0, gs1, ss0, ss1):
    c = lax.axis_index("c")
    s = lax.axis_index("s")
    r0 = s * ROWS_PT
    pltpu.sync_copy(z_hbm.at[pl.ds(r0, ROWS_PT)],
                    acc_sh.at[pl.ds(r0, ROWS_PT)])
    plsc.subcore_barrier()
    _edge_pipeline(h_hbm.at[c], acc_sh, src3_hbm, dst3_hbm, s, NCHT, NBT1,
                   a_s, a_d, b_s, b_d, rows_v, sem_a, sem_b,
                   (gs0, gs1), (ss0, ss1))
    plsc.subcore_barrier()
    pltpu.sync_copy(acc_sh.at[pl.ds(r0, ROWS_PT)],
                    out_hbm.at[pl.ds(c * NPAD + r0, ROWS_PT)])


NPART = 2 * NTILE
EPT2 = E // NPART
NCHT2 = EPT2 // CHUNK
NBT2 = 8
NCHP2 = NBT2 * ITC


@functools.partial(
    pl.kernel,
    out_type=jax.ShapeDtypeStruct((2 * NPAD, OUT), jnp.float32),
    mesh=_mesh,
    scratch_types=[
        pltpu.VMEM((BPB, CHUNK), jnp.int32),
        pltpu.VMEM((BPB, CHUNK), jnp.int32),
        pltpu.VMEM((BPB, CHUNK), jnp.int32),
        pltpu.VMEM((BPB, CHUNK), jnp.int32),
        pltpu.VMEM((2, CHUNK, OUT), jnp.float32),
        pltpu.VMEM_SHARED((NPAD, OUT), jnp.float32),
        pltpu.SemaphoreType.DMA,
        pltpu.SemaphoreType.DMA,
        pltpu.SemaphoreType.DMA,
        pltpu.SemaphoreType.DMA,
        pltpu.SemaphoreType.DMA,
        pltpu.SemaphoreType.DMA,
    ],
)
def _agg_l2(h_hbm, src3_hbm, dst3_hbm, z_hbm, out_hbm,
            a_s, a_d, b_s, b_d, rows_v, acc_sh,
            sem_a, sem_b, gs0, gs1, ss0, ss1):
    c = lax.axis_index("c")
    s = lax.axis_index("s")
    r0 = s * ROWS_PT
    pltpu.sync_copy(z_hbm.at[pl.ds(r0, ROWS_PT)],
                    acc_sh.at[pl.ds(r0, ROWS_PT)])
    plsc.subcore_barrier()
    _edge_pipeline(h_hbm, acc_sh, src3_hbm, dst3_hbm, c * NTILE + s,
                   NCHT2, NBT2, a_s, a_d, b_s, b_d, rows_v, sem_a, sem_b,
                   (gs0, gs1), (ss0, ss1))
    plsc.subcore_barrier()
    pltpu.sync_copy(acc_sh.at[pl.ds(r0, ROWS_PT)],
                    out_hbm.at[pl.ds(c * NPAD + r0, ROWS_PT)])


_BLK = 1000


def _b1_body(x_ref, w_ref, deg_ref, out_ref):
    h = jnp.dot(x_ref[...], w_ref[...], preferred_element_type=jnp.float32)
    ns = lax.rsqrt(jnp.maximum(deg_ref[...], 1.0))
    out_ref[0] = h[:, :HID // 2] * ns
    out_ref[1] = h[:, HID // 2:] * ns


def _b1(x, w1, deg_out):
    return pl.pallas_call(
        _b1_body,
        grid=(N // _BLK,),
        in_specs=[
            pl.BlockSpec((_BLK, IN), lambda i: (i, 0)),
            pl.BlockSpec((IN, HID), lambda i: (0, 0)),
            pl.BlockSpec((_BLK, 1), lambda i: (i, 0)),
        ],
        out_specs=pl.BlockSpec((2, _BLK, HID // 2), lambda i: (0, i, 0)),
        out_shape=jax.ShapeDtypeStruct((2, N, HID // 2), jnp.float32),
    )(x, w1, deg_out)


def _b2_body(a_ref, di_ref, do_ref, b1_ref, w2a_ref, w2b_ref, out_ref):
    nd = lax.rsqrt(jnp.maximum(di_ref[...], 1.0))
    ns = lax.rsqrt(jnp.maximum(do_ref[...], 1.0))
    b = b1_ref[...]
    u0 = jnp.maximum(a_ref[0] * nd + b[:, :HID // 2], 0.0)
    u1 = jnp.maximum(a_ref[1] * nd + b[:, HID // 2:], 0.0)
    h2 = (jnp.dot(u0, w2a_ref[...], preferred_element_type=jnp.float32)
          + jnp.dot(u1, w2b_ref[...], preferred_element_type=jnp.float32))
    out_ref[...] = h2 * ns


def _b2(agg1, deg_in, deg_out, b1, w2a, w2b):
    return pl.pallas_call(
        _b2_body,
        grid=(N // _BLK,),
        in_specs=[
            pl.BlockSpec((2, _BLK, HID // 2), lambda i: (0, i, 0)),
            pl.BlockSpec((_BLK, 1), lambda i: (i, 0)),
            pl.BlockSpec((_BLK, 1), lambda i: (i, 0)),
            pl.BlockSpec((1, HID), lambda i: (0, 0)),
            pl.BlockSpec((HID // 2, OUT), lambda i: (0, 0)),
            pl.BlockSpec((HID // 2, OUT), lambda i: (0, 0)),
        ],
        out_specs=pl.BlockSpec((_BLK, OUT), lambda i: (i, 0)),
        out_shape=jax.ShapeDtypeStruct((N, OUT), jnp.float32),
    )(agg1, deg_in, deg_out, b1, w2a, w2b)


def _b3_body(a_ref, di_ref, b2_ref, out_ref):
    nd = lax.rsqrt(jnp.maximum(di_ref[...], 1.0))
    out_ref[...] = (a_ref[0] + a_ref[1]) * nd + b2_ref[...]


def _b3(agg2, deg_in, b2):
    return pl.pallas_call(
        _b3_body,
        grid=(N // _BLK,),
        in_specs=[
            pl.BlockSpec((2, _BLK, OUT), lambda i: (0, i, 0)),
            pl.BlockSpec((_BLK, 1), lambda i: (i, 0)),
            pl.BlockSpec((1, OUT), lambda i: (0, 0)),
        ],
        out_specs=pl.BlockSpec((_BLK, OUT), lambda i: (i, 0)),
        out_shape=jax.ShapeDtypeStruct((N, OUT), jnp.float32),
    )(agg2, deg_in, b2)


def kernel(features, edge_index, W1, b1, W2, b2):
    src = edge_index[0].astype(jnp.int32)
    dst = edge_index[1].astype(jnp.int32)

    src3_l1 = jnp.pad(src.reshape(NTILE, NCHT, CHUNK),
                      ((0, 0), (0, NCHP1 - NCHT), (0, 0)))
    dst3_l1 = jnp.pad(dst.reshape(NTILE, NCHT, CHUNK),
                      ((0, 0), (0, NCHP1 - NCHT), (0, 0)))
    src3_l2 = jnp.pad(src.reshape(NPART, NCHT2, CHUNK),
                      ((0, 0), (0, NCHP2 - NCHT2), (0, 0)))
    dst3_l2 = jnp.pad(dst.reshape(NPART, NCHT2, CHUNK),
                      ((0, 0), (0, NCHP2 - NCHT2), (0, 0)))

    degs = _deg(jnp.stack([src3_l1, dst3_l1]),
                jnp.zeros((NPAD,), jnp.float32))
    deg_out = degs[0:N].reshape(N, 1)
    deg_in = degs[NPAD:NPAD + N].reshape(N, 1)

    h1 = _b1(features, W1, deg_out)
    agg1 = _agg_l1(h1, src3_l1, dst3_l1,
                   jnp.zeros((NPAD, HID // 2), jnp.float32))

    h2 = _b2(agg1.reshape(2, NPAD, HID // 2), deg_in, deg_out,
             b1.reshape(1, HID), W2[:HID // 2], W2[HID // 2:])
    agg2 = _agg_l2(h2, src3_l2, dst3_l2, jnp.zeros((NPAD, OUT), jnp.float32))

    return _b3(agg2.reshape(2, NPAD, OUT), deg_in, b2.reshape(1, OUT))


# speedup vs baseline: 11.2612x; 1.0925x over previous
import functools

import jax
import jax.numpy as jnp
from jax import lax
from jax.experimental import pallas as pl
from jax.experimental.pallas import tpu as pltpu
from jax.experimental.pallas import tpu_sc as plsc

N = 10000
E = 320000
IN = 128
HID = 256
OUT = 128

NTILE = 16
EPT = E // NTILE
CHUNK = 128
EPTP = 20480
NCH = EPTP // CHUNK
NPAD = 10240
ROWS_PT = NPAD // NTILE

_mesh = plsc.VectorSubcoreMesh(core_axis_name="c", subcore_axis_name="s")


NHIST = 4
DEG_BPB = 8
DEG_ITC = 2 * DEG_BPB
DEG_NBT = 10


@functools.partial(
    pl.kernel,
    out_type=jax.ShapeDtypeStruct((2 * NPAD,), jnp.float32),
    mesh=_mesh,
    scratch_types=[
        pltpu.VMEM((DEG_BPB, CHUNK), jnp.int32),
        pltpu.VMEM((DEG_BPB, CHUNK), jnp.int32),
        pltpu.VMEM((CHUNK,), jnp.float32),
        pltpu.VMEM((NHIST, ROWS_PT), jnp.float32),
        pltpu.VMEM((ROWS_PT,), jnp.float32),
        pltpu.VMEM_SHARED((NPAD,), jnp.float32),
        pltpu.VMEM_SHARED((NPAD,), jnp.float32),
        pltpu.VMEM_SHARED((NPAD,), jnp.float32),
        pltpu.VMEM_SHARED((NPAD,), jnp.float32),
        pltpu.SemaphoreType.DMA,
        pltpu.SemaphoreType.DMA,
        pltpu.SemaphoreType.DMA,
        pltpu.SemaphoreType.DMA,
        pltpu.SemaphoreType.DMA,
        pltpu.SemaphoreType.DMA,
    ],
)
def _deg(edges4_hbm, z_hbm, out_hbm, a_i, b_i, ones_v, part_v, red_v,
         h0, h1, h2, h3, sem_a, sem_b, ss0, ss1, ss2, ss3):
    c = lax.axis_index("c")
    s = lax.axis_index("s")
    hists = (h0, h1, h2, h3)
    ssems = (ss0, ss1, ss2, ss3)
    for k in range(CHUNK // 16):
        ones_v[pl.ds(k * 16, 16)] = jnp.ones((16,), jnp.float32)
    stripe = s * ROWS_PT
    for j in range(NHIST):
        pltpu.sync_copy(z_hbm.at[pl.ds(stripe, ROWS_PT)],
                        hists[j].at[pl.ds(stripe, ROWS_PT)])
    plsc.subcore_barrier()

    def idx_block(buf, row0, sem):
        return pltpu.make_async_copy(edges4_hbm.at[c, s].at[pl.ds(row0,
                                                                  DEG_BPB)],
                                     buf, sem)

    def scat(idx_slice, j):
        return pltpu.make_async_copy(ones_v, hists[j].at[idx_slice],
                                     ssems[j])

    idx_block(a_i, 0, sem_a).start()
    slices = ([a_i.at[i] for i in range(DEG_BPB)]
              + [b_i.at[i] for i in range(DEG_BPB)])

    def body(k, carry):
        g0 = k * DEG_ITC
        idx_block(a_i, g0, sem_a).wait()
        for h in range(DEG_ITC):
            g = g0 + h
            j = h % NHIST
            pd = slices[h - NHIST] if h >= NHIST else slices[DEG_ITC
                                                             - NHIST + h]

            @pl.when(jnp.logical_and(g >= NHIST, g - NHIST < NCH))
            def _(pd=pd, j=j):
                scat(pd, j).wait()

            if h == NHIST:
                idx_block(b_i, g0 + DEG_BPB, sem_b).start()
            if h == DEG_BPB:
                idx_block(b_i, g0 + DEG_BPB, sem_b).wait()
            if h == DEG_BPB + NHIST:
                @pl.when(k + 1 < DEG_NBT)
                def _():
                    idx_block(a_i, g0 + DEG_ITC, sem_a).start()

            ps = slices[h]

            @pl.when(g < NCH)
            def _(ps=ps, j=j):
                scat(ps, j).start(add=True)

        return carry

    lax.fori_loop(0, DEG_NBT, body, 0)
    for t in range(NHIST):
        scat(slices[DEG_ITC - NHIST + t], (NCH - NHIST + t) % NHIST).wait()
    plsc.subcore_barrier()
    for j in range(NHIST):
        pltpu.sync_copy(hists[j].at[pl.ds(stripe, ROWS_PT)],
                        part_v.at[j])

    def red(i, carry):
        sl = pl.ds(i * 16, 16)
        red_v[sl] = (part_v[0, sl] + part_v[1, sl]
                     + part_v[2, sl] + part_v[3, sl])
        return carry

    lax.fori_loop(0, ROWS_PT // 16, red, 0)
    pltpu.sync_copy(red_v, out_hbm.at[pl.ds(c * NPAD + stripe, ROWS_PT)])


BPB = 8
ITC = 2 * BPB


def _edge_pipeline(tab, acc_sh, src3, dst3, p, nch, nbt,
                   a_s, a_d, b_s, b_d, rows_v, sem_a, sem_b, gsems, ssems):
    def idx_block(bufs, bufd, row0, sem):
        return (pltpu.make_async_copy(src3.at[p, pl.ds(row0, BPB)],
                                      bufs, sem),
                pltpu.make_async_copy(dst3.at[p, pl.ds(row0, BPB)],
                                      bufd, sem))

    def start_block(bufs, bufd, row0, sem):
        x, y = idx_block(bufs, bufd, row0, sem)
        x.start()
        y.start()

    def wait_block(bufs, bufd, row0, sem):
        x, y = idx_block(bufs, bufd, row0, sem)
        x.wait()
        y.wait()

    def gath(s_slice, r):
        return pltpu.make_async_copy(tab.at[s_slice], rows_v.at[r],
                                     gsems[r])

    def scat(d_slice, r):
        return pltpu.make_async_copy(rows_v.at[r], acc_sh.at[d_slice],
                                     ssems[r])

    start_block(a_s, a_d, 0, sem_a)
    slices = ([(a_s.at[i], a_d.at[i]) for i in range(BPB)]
              + [(b_s.at[i], b_d.at[i]) for i in range(BPB)])

    def body(k, carry):
        g0 = k * ITC
        wait_block(a_s, a_d, g0, sem_a)
        for h in range(ITC):
            g = g0 + h
            r = h % 2
            if h == 2:
                start_block(b_s, b_d, g0 + BPB, sem_b)
            if h == BPB:
                wait_block(b_s, b_d, g0 + BPB, sem_b)
            if h == BPB + 2:
                @pl.when(k + 1 < nbt)
                def _():
                    start_block(a_s, a_d, g0 + ITC, sem_a)

            ps = slices[h][0]

            @pl.when(g < nch)
            def _(ps=ps, r=r):
                gath(ps, r).start()

            qs, qd = slices[h - 1] if h >= 1 else slices[ITC - 1]
            rq = (h - 1) % 2

            @pl.when(jnp.logical_and(g >= 1, g - 1 < nch))
            def _(qs=qs, qd=qd, rq=rq):
                gath(qs, rq).wait()
                d = scat(qd, rq)
                d.start(add=True)
                d.wait()

        return carry

    lax.fori_loop(0, nbt, body, 0)
    qs, qd = slices[ITC - 1]
    gath(qs, (ITC - 1) % 2).wait()
    d = scat(qd, (ITC - 1) % 2)
    d.start(add=True)
    d.wait()


NCHT = EPTP // CHUNK
NBT1 = 10
NCHP1 = NBT1 * ITC


@functools.partial(
    pl.kernel,
    out_type=jax.ShapeDtypeStruct((2 * NPAD, HID // 2), jnp.float32),
    mesh=_mesh,
    scratch_types=[
        pltpu.VMEM((BPB, CHUNK), jnp.int32),
        pltpu.VMEM((BPB, CHUNK), jnp.int32),
        pltpu.VMEM((BPB, CHUNK), jnp.int32),
        pltpu.VMEM((BPB, CHUNK), jnp.int32),
        pltpu.VMEM((2, CHUNK, HID // 2), jnp.float32),
        pltpu.VMEM_SHARED((NPAD, HID // 2), jnp.float32),
        pltpu.SemaphoreType.DMA,
        pltpu.SemaphoreType.DMA,
        pltpu.SemaphoreType.DMA,
        pltpu.SemaphoreType.DMA,
        pltpu.SemaphoreType.DMA,
        pltpu.SemaphoreType.DMA,
    ],
)
def _agg_l1(h_hbm, src3_hbm, dst3_hbm, z_hbm, out_hbm,
            a_s, a_d, b_s, b_d, rows_v, acc_sh,
            sem_a, sem_b, gs0, gs1, ss0, ss1):
    c = lax.axis_index("c")
    s = lax.axis_index("s")
    r0 = s * ROWS_PT
    pltpu.sync_copy(z_hbm.at[pl.ds(r0, ROWS_PT)],
                    acc_sh.at[pl.ds(r0, ROWS_PT)])
    plsc.subcore_barrier()
    _edge_pipeline(h_hbm.at[c], acc_sh, src3_hbm, dst3_hbm, s, NCHT, NBT1,
                   a_s, a_d, b_s, b_d, rows_v, sem_a, sem_b,
                   (gs0, gs1), (ss0, ss1))
    plsc.subcore_barrier()
    pltpu.sync_copy(acc_sh.at[pl.ds(r0, ROWS_PT)],
                    out_hbm.at[pl.ds(c * NPAD + r0, ROWS_PT)])


NPART = 2 * NTILE
EPT2 = E // NPART
EPT2P = 10240
NCHT2 = EPT2P // CHUNK
NBT2 = 5
NCHP2 = NBT2 * ITC


@functools.partial(
    pl.kernel,
    out_type=jax.ShapeDtypeStruct((2 * NPAD, OUT), jnp.float32),
    mesh=_mesh,
    scratch_types=[
        pltpu.VMEM((BPB, CHUNK), jnp.int32),
        pltpu.VMEM((BPB, CHUNK), jnp.int32),
        pltpu.VMEM((BPB, CHUNK), jnp.int32),
        pltpu.VMEM((BPB, CHUNK), jnp.int32),
        pltpu.VMEM((2, CHUNK, OUT), jnp.float32),
        pltpu.VMEM_SHARED((NPAD, OUT), jnp.float32),
        pltpu.SemaphoreType.DMA,
        pltpu.SemaphoreType.DMA,
        pltpu.SemaphoreType.DMA,
        pltpu.SemaphoreType.DMA,
        pltpu.SemaphoreType.DMA,
        pltpu.SemaphoreType.DMA,
    ],
)
def _agg_l2(h_hbm, src3_hbm, dst3_hbm, z_hbm, out_hbm,
            a_s, a_d, b_s, b_d, rows_v, acc_sh,
            sem_a, sem_b, gs0, gs1, ss0, ss1):
    c = lax.axis_index("c")
    s = lax.axis_index("s")
    r0 = s * ROWS_PT
    pltpu.sync_copy(z_hbm.at[pl.ds(r0, ROWS_PT)],
                    acc_sh.at[pl.ds(r0, ROWS_PT)])
    plsc.subcore_barrier()
    _edge_pipeline(h_hbm, acc_sh, src3_hbm, dst3_hbm, c * NTILE + s,
                   NCHT2, NBT2, a_s, a_d, b_s, b_d, rows_v, sem_a, sem_b,
                   (gs0, gs1), (ss0, ss1))
    plsc.subcore_barrier()
    pltpu.sync_copy(acc_sh.at[pl.ds(r0, ROWS_PT)],
                    out_hbm.at[pl.ds(c * NPAD + r0, ROWS_PT)])


_BLKP = 640
_BLK = 1000


def _b1_body(x_ref, w_ref, deg_ref, out_ref):
    h = jnp.dot(x_ref[...], w_ref[...], preferred_element_type=jnp.float32)
    ns = lax.rsqrt(jnp.maximum(deg_ref[...], 1.0))
    out_ref[0] = h[:, :HID // 2] * ns
    out_ref[1] = h[:, HID // 2:] * ns


def _b1(x, w1, deg_out):
    return pl.pallas_call(
        _b1_body,
        grid=(NPAD // _BLKP,),
        in_specs=[
            pl.BlockSpec((_BLKP, IN), lambda i: (i, 0)),
            pl.BlockSpec((IN, HID), lambda i: (0, 0)),
            pl.BlockSpec((_BLKP, 1), lambda i: (i, 0)),
        ],
        out_specs=pl.BlockSpec((2, _BLKP, HID // 2), lambda i: (0, i, 0)),
        out_shape=jax.ShapeDtypeStruct((2, NPAD, HID // 2), jnp.float32),
    )(x, w1, deg_out)


def _b2_body(a_ref, di_ref, do_ref, b1_ref, w2a_ref, w2b_ref, out_ref):
    nd = lax.rsqrt(jnp.maximum(di_ref[...], 1.0))
    ns = lax.rsqrt(jnp.maximum(do_ref[...], 1.0))
    b = b1_ref[...]
    u0 = jnp.maximum(a_ref[0] * nd + b[:, :HID // 2], 0.0)
    u1 = jnp.maximum(a_ref[1] * nd + b[:, HID // 2:], 0.0)
    h2 = (jnp.dot(u0, w2a_ref[...], preferred_element_type=jnp.float32)
          + jnp.dot(u1, w2b_ref[...], preferred_element_type=jnp.float32))
    out_ref[...] = h2 * ns


def _b2(agg1, deg_in, deg_out, b1, w2a, w2b):
    return pl.pallas_call(
        _b2_body,
        grid=(NPAD // _BLKP,),
        in_specs=[
            pl.BlockSpec((2, _BLKP, HID // 2), lambda i: (0, i, 0)),
            pl.BlockSpec((_BLKP, 1), lambda i: (i, 0)),
            pl.BlockSpec((_BLKP, 1), lambda i: (i, 0)),
            pl.BlockSpec((1, HID), lambda i: (0, 0)),
            pl.BlockSpec((HID // 2, OUT), lambda i: (0, 0)),
            pl.BlockSpec((HID // 2, OUT), lambda i: (0, 0)),
        ],
        out_specs=pl.BlockSpec((_BLKP, OUT), lambda i: (i, 0)),
        out_shape=jax.ShapeDtypeStruct((NPAD, OUT), jnp.float32),
    )(agg1, deg_in, deg_out, b1, w2a, w2b)


def _b3_body(a_ref, di_ref, b2_ref, out_ref):
    nd = lax.rsqrt(jnp.maximum(di_ref[...], 1.0))
    out_ref[...] = (a_ref[0] + a_ref[1]) * nd + b2_ref[...]


def _b3(agg2, deg_in, b2):
    return pl.pallas_call(
        _b3_body,
        grid=(N // _BLK,),
        in_specs=[
            pl.BlockSpec((2, _BLK, OUT), lambda i: (0, i, 0)),
            pl.BlockSpec((_BLK, 1), lambda i: (i, 0)),
            pl.BlockSpec((1, OUT), lambda i: (0, 0)),
        ],
        out_specs=pl.BlockSpec((_BLK, OUT), lambda i: (i, 0)),
        out_shape=jax.ShapeDtypeStruct((N, OUT), jnp.float32),
    )(agg2, deg_in, b2)


def kernel(features, edge_index, W1, b1, W2, b2):
    src = edge_index[0].astype(jnp.int32)
    dst = edge_index[1].astype(jnp.int32)

    pad1 = jnp.broadcast_to(N + (jnp.arange(EPTP - EPT, dtype=jnp.int32)
                                 % (NPAD - N)), (NTILE, EPTP - EPT))
    src3_l1 = jnp.concatenate([src.reshape(NTILE, EPT), pad1],
                              axis=1).reshape(NTILE, NCHT, CHUNK)
    dst3_l1 = jnp.concatenate([dst.reshape(NTILE, EPT), pad1],
                              axis=1).reshape(NTILE, NCHT, CHUNK)
    pad2 = jnp.broadcast_to(N + (jnp.arange(EPT2P - EPT2, dtype=jnp.int32)
                                 % (NPAD - N)), (NPART, EPT2P - EPT2))
    src3_l2 = jnp.concatenate([src.reshape(NPART, EPT2), pad2],
                              axis=1).reshape(NPART, NCHT2, CHUNK)
    dst3_l2 = jnp.concatenate([dst.reshape(NPART, EPT2), pad2],
                              axis=1).reshape(NPART, NCHT2, CHUNK)

    degs = _deg(jnp.stack([src3_l1, dst3_l1]),
                jnp.zeros((NPAD,), jnp.float32))
    deg_out = degs[0:NPAD].reshape(NPAD, 1)
    deg_in = degs[NPAD:].reshape(NPAD, 1)

    xp = jnp.pad(features, ((0, NPAD - N), (0, 0)))
    h1 = _b1(xp, W1, deg_out)
    agg1 = _agg_l1(h1, src3_l1, dst3_l1,
                   jnp.zeros((NPAD, HID // 2), jnp.float32))

    h2 = _b2(agg1.reshape(2, NPAD, HID // 2), deg_in, deg_out,
             b1.reshape(1, HID), W2[:HID // 2], W2[HID // 2:])
    agg2 = _agg_l2(h2, src3_l2, dst3_l2, jnp.zeros((NPAD, OUT), jnp.float32))

    return _b3(agg2.reshape(2, NPAD, OUT), deg_in, b2.reshape(1, OUT))
